# baseline (speedup 1.0000x reference)
import functools

import jax
import jax.numpy as jnp
from jax import lax
from jax.experimental import pallas as pl
from jax.experimental.pallas import tpu as pltpu
from jax.experimental.pallas import tpu_sc as plsc

_NODES = 50000
_EDGES = 800000
_F = 64

_NC, _NS = 2, 16
_NPHASE = 4
_CHUNK = 128
_NCHUNKS = _EDGES // _CHUNK
_QNODES = _NODES // (_NC * _NPHASE)
_ACC_ROWS = 6528
_TILE_ROWS = _ACC_ROWS // _NS
_CNT_N = _ACC_ROWS
_NQ = _NC * _NPHASE

_LN2 = 0.6931471805599453


def _ssp(x):
    return jnp.maximum(x, 0.0) + jnp.log1p(jnp.exp(-jnp.abs(x))) - _LN2


_BE = 2000


def _edge_mlp_body(bf_ref, eh_ref, w1_ref, b1_ref, w2_ref, b2_ref, o_ref):
    h = _ssp(jnp.dot(bf_ref[...], w1_ref[...],
                     preferred_element_type=jnp.float32) + b1_ref[...])
    h = _ssp(jnp.dot(h, w2_ref[...],
                     preferred_element_type=jnp.float32) + b2_ref[...])
    o_ref[...] = h * eh_ref[...]


def _edge_mlp(bf, eh, W1, b1, W2, b2):
    return pl.pallas_call(
        _edge_mlp_body,
        grid=(_EDGES // _BE,),
        in_specs=[
            pl.BlockSpec((_BE, _F), lambda i: (i, 0)),
            pl.BlockSpec((_BE, _F), lambda i: (i, 0)),
            pl.BlockSpec((_F, _F), lambda i: (0, 0)),
            pl.BlockSpec((1, _F), lambda i: (0, 0)),
            pl.BlockSpec((_F, _F), lambda i: (0, 0)),
            pl.BlockSpec((1, _F), lambda i: (0, 0)),
        ],
        out_specs=pl.BlockSpec((_BE, _F), lambda i: (i, 0)),
        out_shape=jax.ShapeDtypeStruct((_EDGES, _F), jnp.float32),
    )(bf, eh, W1, b1, W2, b2)


_EPT = _EDGES // _NS
_FULL_CHUNKS = _EPT // _CHUNK
_REM = _EPT - _FULL_CHUNKS * _CHUNK
_REM_GROUPS = _REM // 16


def _sc_scatter_body(m_hbm, dst_hbm, zrow_hbm, zcnt_hbm, summed_out, counts_out,
                     rows_v, idx_all, idx2_v, pos2_v, cnt_v, acc_sh):
    c = lax.axis_index("c")
    s = lax.axis_index("s")
    ones16 = jnp.ones((16,), jnp.float32)
    iota16 = lax.iota(jnp.int32, 16)
    row0 = s * _TILE_ROWS
    gbase = s * _EPT

    pltpu.sync_copy(dst_hbm.at[pl.ds(s * _EPT, _EPT)], idx_all)

    for p in range(_NPHASE):
        base = c * (_NPHASE * _QNODES) + p * _QNODES

        pltpu.sync_copy(zcnt_hbm, cnt_v)
        for piece in range(3):
            pltpu.sync_copy(zrow_hbm,
                            acc_sh.at[pl.ds(row0 + piece * _CHUNK, _CHUNK)])
        pltpu.sync_copy(zrow_hbm.at[pl.ds(0, 24)],
                        acc_sh.at[pl.ds(row0 + 3 * _CHUNK, 24)])
        plsc.subcore_barrier()

        def _chunk_batch(off, groups):
            for g in range(groups):
                lpos = off + g * 16 + iota16
                v = plsc.load_gather(idx_all, [lpos])
                local = v - base
                inr = jnp.logical_and(local >= 0, local < _QNODES)
                sel = jnp.where(inr, local, _QNODES + (v & 255))
                plsc.addupdate_scatter(cnt_v, [sel], ones16, mask=inr)
                pltpu.sync_copy(rows_v.at[pl.ds(g * 16, 16)],
                                acc_sh.at[sel], add=True)

        def _chunk_body(i, off):
            pltpu.sync_copy(
                m_hbm.at[pl.ds(pl.multiple_of(gbase + off, 16), _CHUNK)],
                rows_v)
            _chunk_batch(off, 8)
            return off + _CHUNK
        lax.fori_loop(0, _FULL_CHUNKS, _chunk_body, 0)

        off_t = _FULL_CHUNKS * _CHUNK
        pltpu.sync_copy(
            m_hbm.at[pl.ds(pl.multiple_of(gbase + off_t, 16), _REM)],
            rows_v.at[pl.ds(0, _REM)])
        _chunk_batch(off_t, _REM_GROUPS)
        plsc.subcore_barrier()

        q = c * _NPHASE + p
        pltpu.sync_copy(acc_sh.at[pl.ds(row0, _TILE_ROWS)],
                        summed_out.at[q, pl.ds(row0, _TILE_ROWS)])
        pltpu.sync_copy(cnt_v.at[pl.ds(0, _ACC_ROWS)], counts_out.at[q, s])
        if p + 1 < _NPHASE:
            plsc.subcore_barrier()


_sc_scatter = functools.partial(
    pl.kernel,
    out_type=(jax.ShapeDtypeStruct((_NQ, _ACC_ROWS, _F), jnp.float32),
              jax.ShapeDtypeStruct((_NQ, _NS, _ACC_ROWS), jnp.float32)),
    mesh=plsc.VectorSubcoreMesh(core_axis_name="c", subcore_axis_name="s",
                                num_cores=_NC, num_subcores=_NS),
    compiler_params=pltpu.CompilerParams(needs_layout_passes=False),
    scratch_types=[
        pltpu.VMEM((_CHUNK, _F), jnp.float32),
        pltpu.VMEM((_EPT,), jnp.int32),
        pltpu.VMEM((1, 128), jnp.int32),
        pltpu.VMEM((1, 128), jnp.int32),
        pltpu.VMEM((_CNT_N,), jnp.float32),
        pltpu.VMEM_SHARED((_ACC_ROWS, _F), jnp.float32),
    ],
)(_sc_scatter_body)


_BN = 128


def _node_mlp_body(s_ref, c_ref, w3_ref, b3_ref, o_ref):
    cnt = jnp.sum(jnp.transpose(c_ref[0]), axis=1, keepdims=True)
    nh = s_ref[0] / jnp.maximum(cnt, 1.0)
    o_ref[0] = _ssp(jnp.dot(nh, w3_ref[...],
                            preferred_element_type=jnp.float32) + b3_ref[...])


def _node_mlp(summed, counts, W3, b3):
    nb = _ACC_ROWS // _BN
    return pl.pallas_call(
        _node_mlp_body,
        grid=(_NQ * nb,),
        in_specs=[
            pl.BlockSpec((1, _BN, _F), lambda i: (i // nb, i % nb, 0)),
            pl.BlockSpec((1, _NS, _BN), lambda i: (i // nb, 0, i % nb)),
            pl.BlockSpec((_F, _F), lambda i: (0, 0)),
            pl.BlockSpec((1, _F), lambda i: (0, 0)),
        ],
        out_specs=pl.BlockSpec((1, _BN, _F), lambda i: (i // nb, i % nb, 0)),
        out_shape=jax.ShapeDtypeStruct((_NQ, _ACC_ROWS, _F), jnp.float32),
    )(summed, counts, W3, b3)


def kernel(bf, eh, edge_index, W1, b1, W2, b2, W3, b3):
    dst = edge_index[1].astype(jnp.int32)
    m = _edge_mlp(bf, eh, W1, b1.reshape(1, _F), W2, b2.reshape(1, _F))
    zrow = jnp.zeros((_CHUNK, _F), jnp.float32)
    zcnt = jnp.zeros((_CNT_N,), jnp.float32)
    summed, counts = _sc_scatter(m, dst, zrow, zcnt)
    if True:
        ss = jax.ops.segment_sum(m, dst, num_segments=_NODES)
        ss = ss.reshape(_NQ, _QNODES, _F)
        summed = jnp.concatenate(
            [ss, jnp.zeros((_NQ, _ACC_ROWS - _QNODES, _F), jnp.float32)],
            axis=1)
    out_padded = _node_mlp(summed, counts, W3, b3.reshape(1, _F))
    return out_padded[:, :_QNODES, :].reshape(_NODES, _F)


# speedup vs baseline: 1.8364x; 1.8364x over previous
import functools

import jax
import jax.numpy as jnp
from jax import lax
from jax.experimental import pallas as pl
from jax.experimental.pallas import tpu as pltpu
from jax.experimental.pallas import tpu_sc as plsc

_NODES = 50000
_EDGES = 800000
_F = 64

_NC, _NS = 2, 16
_HALF = _NODES // _NC
_CNT_N = 25600

_LN2 = 0.6931471805599453


def _ssp(x):
    return jnp.maximum(x, 0.0) + jnp.log1p(jnp.exp(-jnp.abs(x))) - _LN2


_BE = 2000


def _edge_mlp_body(bf_ref, eh_ref, w1_ref, b1_ref, w2_ref, b2_ref, o_ref):
    h = _ssp(jnp.dot(bf_ref[...], w1_ref[...],
                     preferred_element_type=jnp.float32) + b1_ref[...])
    h = _ssp(jnp.dot(h, w2_ref[...],
                     preferred_element_type=jnp.float32) + b2_ref[...])
    o_ref[...] = h * eh_ref[...]


def _edge_mlp(bf, eh, W1, b1, W2, b2):
    return pl.pallas_call(
        _edge_mlp_body,
        grid=(_EDGES // _BE,),
        in_specs=[
            pl.BlockSpec((_BE, _F), lambda i: (i, 0)),
            pl.BlockSpec((_BE, _F), lambda i: (i, 0)),
            pl.BlockSpec((_F, _F), lambda i: (0, 0)),
            pl.BlockSpec((1, _F), lambda i: (0, 0)),
            pl.BlockSpec((_F, _F), lambda i: (0, 0)),
            pl.BlockSpec((1, _F), lambda i: (0, 0)),
        ],
        out_specs=pl.BlockSpec((_BE, _F), lambda i: (i, 0)),
        out_shape=jax.ShapeDtypeStruct((_EDGES, _F), jnp.float32),
    )(bf, eh, W1, b1, W2, b2)


_EPT = _EDGES // _NS
_CHUNK = 128
_FULL_CHUNKS = _EPT // _CHUNK
_REM = _EPT - _FULL_CHUNKS * _CHUNK
_REM_GROUPS = _REM // 16


def _sc_counts_body(dst_hbm, zcnt_hbm, counts_out, idx_all, cnt_v):
    c = lax.axis_index("c")
    s = lax.axis_index("s")
    ones16 = jnp.ones((16,), jnp.float32)
    iota16 = lax.iota(jnp.int32, 16)
    base = c * _HALF

    pltpu.sync_copy(dst_hbm.at[pl.ds(s * _EPT, _EPT)], idx_all)
    pltpu.sync_copy(zcnt_hbm, cnt_v)

    def _groups(off, groups):
        for g in range(groups):
            lpos = off + g * 16 + iota16
            v = plsc.load_gather(idx_all, [lpos])
            local = v - base
            inr = jnp.logical_and(local >= 0, local < _HALF)
            sel = jnp.where(inr, local, _HALF + (v & 255))
            plsc.addupdate_scatter(cnt_v, [sel], ones16, mask=inr)

    def _chunk_body(i, off):
        _groups(off, 8)
        return off + _CHUNK
    lax.fori_loop(0, _FULL_CHUNKS, _chunk_body, 0)
    _groups(_FULL_CHUNKS * _CHUNK, _REM_GROUPS)

    pltpu.sync_copy(cnt_v, counts_out.at[c, s])


_sc_counts = functools.partial(
    pl.kernel,
    out_type=jax.ShapeDtypeStruct((_NC, _NS, _CNT_N), jnp.float32),
    mesh=plsc.VectorSubcoreMesh(core_axis_name="c", subcore_axis_name="s",
                                num_cores=_NC, num_subcores=_NS),
    compiler_params=pltpu.CompilerParams(needs_layout_passes=False),
    scratch_types=[
        pltpu.VMEM((_EPT,), jnp.int32),
        pltpu.VMEM((_CNT_N,), jnp.float32),
    ],
)(_sc_counts_body)


_BN = 128


def _node_mlp_body(s_ref, c_ref, w3_ref, b3_ref, o_ref):
    cnt = jnp.sum(jnp.transpose(c_ref[0]), axis=1, keepdims=True)
    nh = s_ref[0] / jnp.maximum(cnt, 1.0)
    o_ref[0] = _ssp(jnp.dot(nh, w3_ref[...],
                            preferred_element_type=jnp.float32) + b3_ref[...])


def _node_mlp(summed, counts, W3, b3):
    nb = _CNT_N // _BN
    return pl.pallas_call(
        _node_mlp_body,
        grid=(_NC * nb,),
        in_specs=[
            pl.BlockSpec((1, _BN, _F), lambda i: (i // nb, i % nb, 0)),
            pl.BlockSpec((1, _NS, _BN), lambda i: (i // nb, 0, i % nb)),
            pl.BlockSpec((_F, _F), lambda i: (0, 0)),
            pl.BlockSpec((1, _F), lambda i: (0, 0)),
        ],
        out_specs=pl.BlockSpec((1, _BN, _F), lambda i: (i // nb, i % nb, 0)),
        out_shape=jax.ShapeDtypeStruct((_NC, _CNT_N, _F), jnp.float32),
    )(summed, counts, W3, b3)


def kernel(bf, eh, edge_index, W1, b1, W2, b2, W3, b3):
    dst = edge_index[1].astype(jnp.int32)
    m = _edge_mlp(bf, eh, W1, b1.reshape(1, _F), W2, b2.reshape(1, _F))
    zcnt = jnp.zeros((_CNT_N,), jnp.float32)
    counts = _sc_counts(dst, zcnt)
    summed = jax.ops.segment_sum(m, dst, num_segments=_NODES)
    summed = jnp.concatenate(
        [summed.reshape(_NC, _HALF, _F),
         jnp.zeros((_NC, _CNT_N - _HALF, _F), jnp.float32)], axis=1)
    out_padded = _node_mlp(summed, counts, W3, b3.reshape(1, _F))
    return out_padded[:, :_HALF, :].reshape(_NODES, _F)
